# baseline (speedup 1.0000x reference)
import functools
import jax
import jax.numpy as jnp
from jax import lax
from jax.experimental import pallas as pl
from jax.experimental.pallas import tpu as pltpu
from jax.experimental.pallas import tpu_sc as plsc


def _gcn_body(adj_ref, x_ref, w_ref, b_ref, o_ref):
    ax = jnp.dot(adj_ref[...], x_ref[...],
                 preferred_element_type=jnp.float32)
    o_ref[...] = jnp.dot(ax, w_ref[...],
                         preferred_element_type=jnp.float32) + b_ref[...]


_ROWS_PER_W = 40
_CHUNK = 4


def _sc_stream_body(adj_hbm, out_hbm, buf0, buf1, sem0, sem1):
    info = plsc.get_sparse_core_info()
    nc = info.num_cores
    wid = lax.axis_index("s") * nc + lax.axis_index("c")
    base = wid * _ROWS_PER_W
    bufs = (buf0, buf1)
    sems = (sem0, sem1)
    n_chunks = _ROWS_PER_W // _CHUNK
    for c in range(n_chunks):
        cp = pltpu.make_async_copy(
            adj_hbm.at[pl.ds(base + c * _CHUNK, _CHUNK), :],
            bufs[c % 2], sems[c % 2])
        cp.start()
        if c >= 1:
            pltpu.make_async_copy(
                adj_hbm.at[pl.ds(base + (c - 1) * _CHUNK, _CHUNK), :],
                bufs[(c - 1) % 2], sems[(c - 1) % 2]).wait()
    pltpu.make_async_copy(
        adj_hbm.at[pl.ds(base + (n_chunks - 1) * _CHUNK, _CHUNK), :],
        bufs[(n_chunks - 1) % 2], sems[(n_chunks - 1) % 2]).wait()
    pltpu.sync_copy(buf0.at[0, pl.ds(0, 16)], out_hbm.at[wid])


def kernel(input, adj, weight, bias):
    n, d_in = input.shape
    d_out = weight.shape[1]

    mesh = plsc.VectorSubcoreMesh(core_axis_name="c", subcore_axis_name="s")
    sc_probe = functools.partial(
        pl.kernel,
        mesh=mesh,
        out_type=jax.ShapeDtypeStruct((32, 16), jnp.float32),
        scratch_types=[
            pltpu.VMEM((_CHUNK, n), jnp.float32),
            pltpu.VMEM((_CHUNK, n), jnp.float32),
            pltpu.SemaphoreType.DMA,
            pltpu.SemaphoreType.DMA,
        ],
    )(_sc_stream_body)
    dummy = sc_probe(adj)

    tm = 400
    out = pl.pallas_call(
        _gcn_body,
        grid=(n // tm,),
        in_specs=[
            pl.BlockSpec((tm, n), lambda i: (i, 0)),
            pl.BlockSpec((n, d_in), lambda i: (0, 0)),
            pl.BlockSpec((d_in, d_out), lambda i: (0, 0)),
            pl.BlockSpec((1, d_out), lambda i: (0, 0)),
        ],
        out_specs=pl.BlockSpec((tm, d_out), lambda i: (i, 0)),
        out_shape=jax.ShapeDtypeStruct((n, d_out), jnp.float32),
        compiler_params=pltpu.CompilerParams(
            dimension_semantics=("parallel",)),
    )(adj, input, weight, bias.reshape(1, d_out))
    return out, dummy


# speedup vs baseline: 1.2695x; 1.2695x over previous
import jax
import jax.numpy as jnp
from jax.experimental import pallas as pl
from jax.experimental.pallas import tpu as pltpu


def _gcn_body(adj_ref, x_ref, w_ref, b_ref, o_ref):
    ax = jnp.dot(adj_ref[...].astype(jnp.bfloat16),
                 x_ref[...].astype(jnp.bfloat16),
                 preferred_element_type=jnp.float32)
    o_ref[...] = jnp.dot(ax, w_ref[...],
                         preferred_element_type=jnp.float32) + b_ref[...]


def kernel(input, adj, weight, bias):
    n, d_in = input.shape
    d_out = weight.shape[1]

    tm = 400
    out = pl.pallas_call(
        _gcn_body,
        grid=(n // tm,),
        in_specs=[
            pl.BlockSpec((tm, n), lambda i: (i, 0)),
            pl.BlockSpec((n, d_in), lambda i: (0, 0)),
            pl.BlockSpec((d_in, d_out), lambda i: (0, 0)),
            pl.BlockSpec((1, d_out), lambda i: (0, 0)),
        ],
        out_specs=pl.BlockSpec((tm, d_out), lambda i: (i, 0)),
        out_shape=jax.ShapeDtypeStruct((n, d_out), jnp.float32),
        compiler_params=pltpu.CompilerParams(
            dimension_semantics=("parallel",)),
    )(adj, input, weight, bias.reshape(1, d_out))
    return out
